# baseline (speedup 1.0000x reference)
import functools

import jax
import jax.numpy as jnp
from jax import lax
from jax.experimental import pallas as pl
from jax.experimental.pallas import tpu as pltpu
from jax.experimental.pallas import tpu_sc as plsc

_K = 512
_L = 16
_U = 8
_UC = 4


def _batched(nvec, loads, compute, stores):
    def wrap(i, c):
        vals = [loads(i * _U + k) for k in range(_U)]
        outs = [compute(v) for v in vals]
        for k in range(_U):
            stores(i * _U + k, outs[k])
        return c

    lax.fori_loop(0, nvec // _U, wrap, 0)


def _scan_desc(hist, nvregs, carry0, kk):
    iota = lax.iota(jnp.int32, _L)
    z = jnp.zeros((_L,), jnp.int32)

    def l1wrap(i, st):
        for k in range(_U):
            carry, found, jc, cat = st
            j = nvregs - 1 - (i * _U + k)
            t = jnp.sum(hist[pl.ds(j * _L, _L)])
            crossed = jnp.logical_and(found == 0, carry + t >= kk)
            jc = jnp.where(crossed, j, jc)
            cat = jnp.where(crossed, carry, cat)
            found = jnp.where(crossed, jnp.int32(1), found)
            st = (carry + t, found, jc, cat)
        return st

    st = (carry0, jnp.int32(0), jnp.int32(0), carry0)
    _, _, jc, cat = lax.fori_loop(0, nvregs // _U, l1wrap, st)

    h = hist[pl.ds(jc * _L, _L)]
    s = lax.rev(plsc.cumsum(lax.rev(h, (0,))), (0,))
    ge = (cat + s) >= kk
    cnt_ge = jnp.sum(ge.astype(jnp.int32))
    lc = cnt_ge - 1
    sl = jnp.sum(jnp.where(iota == lc, s, 0))
    hl = jnp.sum(jnp.where(iota == lc, h, 0))
    bkt = jc * _L + lc
    cabove = cat + sl - hl

    def zwrap(i, c):
        for k in range(_U):
            hist[pl.ds((i * _U + k) * _L, _L)] = z
        return c

    lax.fori_loop(0, nvregs // _U, zwrap, 0)
    return bkt, cabove


def _sc_topk_body(x_hbm, o_hbm, buf0, buf1, cbuf, hist, si0, si1, so0, so1):
    cid = lax.axis_index("c")
    sid = lax.axis_index("s")
    wid = sid * 2 + cid
    nvec = buf0.shape[0] // _L
    ones = jnp.ones((_L,), jnp.int32)
    zeros = jnp.zeros((_L,), jnp.int32)
    lanes = lax.iota(jnp.int32, _L)
    top = jnp.uint32(0x80000000)
    lowm = jnp.uint32(0x1FFFFF)

    r0 = wid * 2
    r1 = r0 + 1
    in0 = pltpu.async_copy(x_hbm.at[r0], buf0, si0)
    in1 = pltpu.async_copy(x_hbm.at[r1], buf1, si1)

    def zbody(i, c):
        for k in range(_U):
            hist[pl.ds((i * _U + k) * _L, _L)] = zeros
        return c

    lax.fori_loop(0, (2048 // _L) // _U, zbody, 0)

    def do_row(buf):
        def pa_compute(v):
            u = lax.bitcast_convert_type(v, jnp.uint32)
            key = jnp.where(u >= top, ~u, u | top)
            return key, (key >> 21).astype(jnp.int32)

        def pa_store(i, out):
            key, b = out
            buf[pl.ds(i * _L, _L)] = lax.bitcast_convert_type(key,
                                                              jnp.float32)
            plsc.addupdate_scatter(hist, [b], ones)

        _batched(nvec, lambda i: buf[pl.ds(i * _L, _L)], pa_compute,
                 pa_store)
        b1, ca1 = _scan_desc(hist, 2048 // _L, jnp.int32(0), _K)
        b1u = b1.astype(jnp.uint32)

        def cwrap(i, off):
            vs = [buf[pl.ds((i * _U + k) * _L, _L)] for k in range(_U)]
            outs = []
            for v in vs:
                key = lax.bitcast_convert_type(v, jnp.uint32)
                match = (key >> 21) == b1u
                low = (key & lowm).astype(jnp.int32)
                ceq = plsc.cumsum(match.astype(jnp.int32))
                outs.append((match, low, ceq))
            for match, low, ceq in outs:
                plsc.store_scatter(cbuf, [off + ceq - 1], low, mask=match)
                off = off + jnp.sum(match.astype(jnp.int32))
            return off

        off = lax.fori_loop(0, nvec // _U, cwrap, jnp.int32(0))

        last = off // _L
        tmask = lanes < (off - last * _L)
        tv = cbuf[pl.ds(last * _L, _L)]
        cbuf[pl.ds(last * _L, _L)] = jnp.where(tmask, tv, 0)
        for d in range(1, _UC):
            cbuf[pl.ds((last + d) * _L, _L)] = zeros
        nblk = (off + _UC * _L - 1) // (_UC * _L)

        kk2 = jnp.int32(_K) - ca1

        def count_ge(cand):
            def body(s, acc):
                a = acc
                for k in range(_UC):
                    v = cbuf[pl.ds((s * _UC + k) * _L, _L)]
                    a = a + (v >= cand).astype(jnp.int32)
                return a
            acc = lax.fori_loop(0, nblk, body, zeros)
            return jnp.sum(acc)

        t = jnp.int32(0)
        for bit in range(20, -1, -1):
            cand = t | jnp.int32(1 << bit)
            t = jnp.where(count_ge(cand) >= kk2, cand, t)

        def stats_body(s, st):
            ag, ae = st
            for k in range(_UC):
                v = cbuf[pl.ds((s * _UC + k) * _L, _L)]
                ag = ag + (v > t).astype(jnp.int32)
                ae = ae + (v == t).astype(jnp.int32)
            return ag, ae

        ag, ae = lax.fori_loop(0, nblk, stats_body, (zeros, zeros))
        count_gt = ca1 + jnp.sum(ag)
        ebin = jnp.sum(ae)
        m = jnp.int32(_K) - count_gt
        thr = (b1u << 21) | t.astype(jnp.uint32)

        def unkey(key):
            u = jnp.where(key >= top, key ^ top, ~key)
            return lax.bitcast_convert_type(u, jnp.float32)

        @pl.when(m == ebin)
        def _simple():
            def po_compute(v):
                key = lax.bitcast_convert_type(v, jnp.uint32)
                return jnp.where(key >= thr, unkey(key), jnp.float32(0.0))

            def po_store(i, out):
                buf[pl.ds(i * _L, _L)] = out

            _batched(nvec, lambda i: buf[pl.ds(i * _L, _L)], po_compute,
                     po_store)

        @pl.when(m != ebin)
        def _ties():
            def po(i, eqc):
                key = lax.bitcast_convert_type(buf[pl.ds(i * _L, _L)],
                                               jnp.uint32)
                gt = key > thr
                eq = key == thr
                ceq = plsc.cumsum(eq.astype(jnp.int32))
                keep = gt | (eq & ((eqc + ceq) <= m))
                buf[pl.ds(i * _L, _L)] = jnp.where(keep, unkey(key),
                                                   jnp.float32(0.0))
                return eqc + jnp.sum(eq.astype(jnp.int32))

            lax.fori_loop(0, nvec, po, jnp.int32(0))

    in0.wait()
    do_row(buf0)
    out0 = pltpu.async_copy(buf0, o_hbm.at[r0], so0)
    in1.wait()
    do_row(buf1)
    out1 = pltpu.async_copy(buf1, o_hbm.at[r1], so1)
    out0.wait()
    out1.wait()


@jax.jit
def kernel(x):
    b, n = x.shape
    mesh = plsc.VectorSubcoreMesh(
        core_axis_name="c", subcore_axis_name="s", num_cores=2,
        num_subcores=16)
    run = functools.partial(
        pl.kernel,
        out_type=jax.ShapeDtypeStruct((b, n), jnp.float32),
        mesh=mesh,
        compiler_params=pltpu.CompilerParams(needs_layout_passes=False),
        scratch_types=[
            pltpu.VMEM((n,), jnp.float32),
            pltpu.VMEM((n,), jnp.float32),
            pltpu.VMEM((n + _UC * _L,), jnp.int32),
            pltpu.VMEM((2048,), jnp.int32),
            pltpu.SemaphoreType.DMA,
            pltpu.SemaphoreType.DMA,
            pltpu.SemaphoreType.DMA,
            pltpu.SemaphoreType.DMA,
        ],
    )(_sc_topk_body)
    return run(x)


# speedup vs baseline: 1.1180x; 1.1180x over previous
import functools

import jax
import jax.numpy as jnp
from jax import lax
from jax.experimental import pallas as pl
from jax.experimental.pallas import tpu as pltpu
from jax.experimental.pallas import tpu_sc as plsc

_K = 512
_L = 16
_U = 8


def _batched(nvec, loads, compute, stores, u=_U):
    def wrap(i, c):
        vals = [loads(i * u + k) for k in range(u)]
        outs = [compute(v) for v in vals]
        for k in range(u):
            stores(i * u + k, outs[k])
        return c

    lax.fori_loop(0, nvec // u, wrap, 0)


def _scan_desc(hist, nvregs, carry0, kk):
    iota = lax.iota(jnp.int32, _L)
    z = jnp.zeros((_L,), jnp.int32)

    def l1wrap(i, st):
        for k in range(_U):
            carry, found, jc, cat = st
            j = nvregs - 1 - (i * _U + k)
            t = jnp.sum(hist[pl.ds(j * _L, _L)])
            crossed = jnp.logical_and(found == 0, carry + t >= kk)
            jc = jnp.where(crossed, j, jc)
            cat = jnp.where(crossed, carry, cat)
            found = jnp.where(crossed, jnp.int32(1), found)
            st = (carry + t, found, jc, cat)
        return st

    st = (carry0, jnp.int32(0), jnp.int32(0), carry0)
    _, _, jc, cat = lax.fori_loop(0, nvregs // _U, l1wrap, st)

    h = hist[pl.ds(jc * _L, _L)]
    s = lax.rev(plsc.cumsum(lax.rev(h, (0,))), (0,))
    ge = (cat + s) >= kk
    cnt_ge = jnp.sum(ge.astype(jnp.int32))
    lc = cnt_ge - 1
    sl = jnp.sum(jnp.where(iota == lc, s, 0))
    hl = jnp.sum(jnp.where(iota == lc, h, 0))
    bkt = jc * _L + lc
    cabove = cat + sl - hl

    def zwrap(i, c):
        for k in range(_U):
            hist[pl.ds((i * _U + k) * _L, _L)] = z
        return c

    lax.fori_loop(0, nvregs // _U, zwrap, 0)
    return bkt, cabove, hl


def _sc_topk_body(x_hbm, o_hbm, buf0, buf1, hist, si0, si1, so0, so1):
    cid = lax.axis_index("c")
    sid = lax.axis_index("s")
    wid = sid * 2 + cid
    nvec = buf0.shape[0] // _L
    ones = jnp.ones((_L,), jnp.int32)
    top = jnp.uint32(0x80000000)

    r0 = wid * 2
    r1 = r0 + 1
    in0 = pltpu.async_copy(x_hbm.at[r0], buf0, si0)
    in1 = pltpu.async_copy(x_hbm.at[r1], buf1, si1)

    z = jnp.zeros((_L,), jnp.int32)

    def zbody(i, c):
        for k in range(_U):
            hist[pl.ds((i * _U + k) * _L, _L)] = z
        return c

    lax.fori_loop(0, (2048 // _L) // _U, zbody, 0)

    def do_row(buf):
        def pa_compute(v):
            u = lax.bitcast_convert_type(v, jnp.uint32)
            key = jnp.where(u >= top, ~u, u | top)
            return key, (key >> 21).astype(jnp.int32)

        def pa_store(i, out):
            key, b = out
            buf[pl.ds(i * _L, _L)] = lax.bitcast_convert_type(key,
                                                              jnp.float32)
            plsc.addupdate_scatter(hist, [b], ones)

        _batched(nvec, lambda i: buf[pl.ds(i * _L, _L)], pa_compute,
                 pa_store)
        b1, ca1, _ = _scan_desc(hist, 2048 // _L, jnp.int32(0), _K)

        b1u = b1.astype(jnp.uint32)

        def pb_compute(v):
            key = lax.bitcast_convert_type(v, jnp.uint32)
            match = (key >> 21) == b1u
            return ((key >> 10) & jnp.uint32(0x7FF)).astype(jnp.int32), match

        def pb_store(i, out):
            b, match = out
            plsc.addupdate_scatter(hist, [b], ones, mask=match)

        _batched(nvec, lambda i: buf[pl.ds(i * _L, _L)], pb_compute,
                 pb_store, u=16)
        b2, ca2, _ = _scan_desc(hist, 2048 // _L, ca1, _K)

        pref2 = (b1u << 11) | b2.astype(jnp.uint32)

        def pc_compute(v):
            key = lax.bitcast_convert_type(v, jnp.uint32)
            match = (key >> 10) == pref2
            return (key & jnp.uint32(0x3FF)).astype(jnp.int32), match

        def pc_store(i, out):
            b, match = out
            plsc.addupdate_scatter(hist, [b], ones, mask=match)

        _batched(nvec, lambda i: buf[pl.ds(i * _L, _L)], pc_compute,
                 pc_store, u=16)
        b3, ca3, ebin = _scan_desc(hist, 1024 // _L, ca2, _K)

        thr = (pref2 << 10) | b3.astype(jnp.uint32)
        m = jnp.int32(_K) - ca3

        def unkey(key):
            u = jnp.where(key >= top, key ^ top, ~key)
            return lax.bitcast_convert_type(u, jnp.float32)

        @pl.when(m == ebin)
        def _simple():
            def po_compute(v):
                key = lax.bitcast_convert_type(v, jnp.uint32)
                return jnp.where(key >= thr, unkey(key), jnp.float32(0.0))

            def po_store(i, out):
                buf[pl.ds(i * _L, _L)] = out

            _batched(nvec, lambda i: buf[pl.ds(i * _L, _L)], po_compute,
                     po_store, u=16)

        @pl.when(m != ebin)
        def _ties():
            def po(i, eqc):
                key = lax.bitcast_convert_type(buf[pl.ds(i * _L, _L)],
                                               jnp.uint32)
                gt = key > thr
                eq = key == thr
                ceq = plsc.cumsum(eq.astype(jnp.int32))
                keep = gt | (eq & ((eqc + ceq) <= m))
                buf[pl.ds(i * _L, _L)] = jnp.where(keep, unkey(key),
                                                   jnp.float32(0.0))
                return eqc + jnp.sum(eq.astype(jnp.int32))

            lax.fori_loop(0, nvec, po, jnp.int32(0))

    in0.wait()
    do_row(buf0)
    out0 = pltpu.async_copy(buf0, o_hbm.at[r0], so0)
    in1.wait()
    do_row(buf1)
    out1 = pltpu.async_copy(buf1, o_hbm.at[r1], so1)
    out0.wait()
    out1.wait()


@jax.jit
def kernel(x):
    b, n = x.shape
    mesh = plsc.VectorSubcoreMesh(
        core_axis_name="c", subcore_axis_name="s", num_cores=2,
        num_subcores=16)
    run = functools.partial(
        pl.kernel,
        out_type=jax.ShapeDtypeStruct((b, n), jnp.float32),
        mesh=mesh,
        compiler_params=pltpu.CompilerParams(needs_layout_passes=False),
        scratch_types=[
            pltpu.VMEM((n,), jnp.float32),
            pltpu.VMEM((n,), jnp.float32),
            pltpu.VMEM((2048,), jnp.int32),
            pltpu.SemaphoreType.DMA,
            pltpu.SemaphoreType.DMA,
            pltpu.SemaphoreType.DMA,
            pltpu.SemaphoreType.DMA,
        ],
    )(_sc_topk_body)
    return run(x)


# speedup vs baseline: 1.1396x; 1.0193x over previous
import functools

import jax
import jax.numpy as jnp
from jax import lax
from jax.experimental import pallas as pl
from jax.experimental.pallas import tpu as pltpu
from jax.experimental.pallas import tpu_sc as plsc

_K = 512
_L = 16
_U = 8


def _batched(nvec, loads, compute, stores, u=_U):
    def wrap(i, c):
        vals = [loads(i * u + k) for k in range(u)]
        outs = [compute(v) for v in vals]
        for k in range(u):
            stores(i * u + k, outs[k])
        return c

    lax.fori_loop(0, nvec // u, wrap, 0)


def _scan_desc(hist, nvregs, carry0, kk):
    iota = lax.iota(jnp.int32, _L)
    z = jnp.zeros((_L,), jnp.int32)

    def l1wrap(i, st):
        for k in range(_U):
            carry, found, jc, cat = st
            j = nvregs - 1 - (i * _U + k)
            t = jnp.sum(hist[pl.ds(j * _L, _L)])
            crossed = jnp.logical_and(found == 0, carry + t >= kk)
            jc = jnp.where(crossed, j, jc)
            cat = jnp.where(crossed, carry, cat)
            found = jnp.where(crossed, jnp.int32(1), found)
            st = (carry + t, found, jc, cat)
        return st

    st = (carry0, jnp.int32(0), jnp.int32(0), carry0)
    _, _, jc, cat = lax.fori_loop(0, nvregs // _U, l1wrap, st)

    h = hist[pl.ds(jc * _L, _L)]
    s = lax.rev(plsc.cumsum(lax.rev(h, (0,))), (0,))
    ge = (cat + s) >= kk
    cnt_ge = jnp.sum(ge.astype(jnp.int32))
    lc = cnt_ge - 1
    sl = jnp.sum(jnp.where(iota == lc, s, 0))
    hl = jnp.sum(jnp.where(iota == lc, h, 0))
    bkt = jc * _L + lc
    cabove = cat + sl - hl

    def zwrap(i, c):
        for k in range(_U):
            hist[pl.ds((i * _U + k) * _L, _L)] = z
        return c

    lax.fori_loop(0, nvregs // _U, zwrap, 0)
    return bkt, cabove, hl


def _sc_topk_body(x_hbm, o_hbm, buf0, buf1, hist, si0, si1, so0, so1):
    cid = lax.axis_index("c")
    sid = lax.axis_index("s")
    wid = sid * 2 + cid
    nvec = buf0.shape[0] // _L
    ones = jnp.ones((_L,), jnp.int32)
    top = jnp.uint32(0x80000000)

    r0 = wid * 2
    r1 = r0 + 1
    in0 = pltpu.async_copy(x_hbm.at[r0], buf0, si0)
    in1 = pltpu.async_copy(x_hbm.at[r1], buf1, si1)

    z = jnp.zeros((_L,), jnp.int32)

    def zbody(i, c):
        for k in range(_U):
            hist[pl.ds((i * _U + k) * _L, _L)] = z
        return c

    lax.fori_loop(0, (2048 // _L) // _U, zbody, 0)

    def do_row(buf):
        def pa_compute(v):
            u = lax.bitcast_convert_type(v, jnp.uint32)
            key = jnp.where(u >= top, ~u, u | top)
            return key, (key >> 21).astype(jnp.int32)

        def pa_store(i, out):
            key, b = out
            buf[pl.ds(i * _L, _L)] = lax.bitcast_convert_type(key,
                                                              jnp.float32)
            plsc.addupdate_scatter(hist, [b], ones)

        _batched(nvec, lambda i: buf[pl.ds(i * _L, _L)], pa_compute,
                 pa_store, u=16)
        b1, ca1, _ = _scan_desc(hist, 2048 // _L, jnp.int32(0), _K)

        b1u = b1.astype(jnp.uint32)

        def pb_compute(v):
            key = lax.bitcast_convert_type(v, jnp.uint32)
            match = (key >> 21) == b1u
            return ((key >> 10) & jnp.uint32(0x7FF)).astype(jnp.int32), match

        def pb_store(i, out):
            b, match = out
            plsc.addupdate_scatter(hist, [b], ones, mask=match)

        _batched(nvec, lambda i: buf[pl.ds(i * _L, _L)], pb_compute,
                 pb_store, u=16)
        b2, ca2, _ = _scan_desc(hist, 2048 // _L, ca1, _K)

        pref2 = (b1u << 11) | b2.astype(jnp.uint32)

        def pc_compute(v):
            key = lax.bitcast_convert_type(v, jnp.uint32)
            match = (key >> 10) == pref2
            return (key & jnp.uint32(0x3FF)).astype(jnp.int32), match

        def pc_store(i, out):
            b, match = out
            plsc.addupdate_scatter(hist, [b], ones, mask=match)

        _batched(nvec, lambda i: buf[pl.ds(i * _L, _L)], pc_compute,
                 pc_store, u=16)
        b3, ca3, ebin = _scan_desc(hist, 1024 // _L, ca2, _K)

        thr = (pref2 << 10) | b3.astype(jnp.uint32)
        m = jnp.int32(_K) - ca3

        def unkey(key):
            u = jnp.where(key >= top, key ^ top, ~key)
            return lax.bitcast_convert_type(u, jnp.float32)

        @pl.when(m == ebin)
        def _simple():
            def po_compute(v):
                key = lax.bitcast_convert_type(v, jnp.uint32)
                return jnp.where(key >= thr, unkey(key), jnp.float32(0.0))

            def po_store(i, out):
                buf[pl.ds(i * _L, _L)] = out

            _batched(nvec, lambda i: buf[pl.ds(i * _L, _L)], po_compute,
                     po_store, u=16)

        @pl.when(m != ebin)
        def _ties():
            def po(i, eqc):
                key = lax.bitcast_convert_type(buf[pl.ds(i * _L, _L)],
                                               jnp.uint32)
                gt = key > thr
                eq = key == thr
                ceq = plsc.cumsum(eq.astype(jnp.int32))
                keep = gt | (eq & ((eqc + ceq) <= m))
                buf[pl.ds(i * _L, _L)] = jnp.where(keep, unkey(key),
                                                   jnp.float32(0.0))
                return eqc + jnp.sum(eq.astype(jnp.int32))

            lax.fori_loop(0, nvec, po, jnp.int32(0))

    in0.wait()
    do_row(buf0)
    out0 = pltpu.async_copy(buf0, o_hbm.at[r0], so0)
    in1.wait()
    do_row(buf1)
    out1 = pltpu.async_copy(buf1, o_hbm.at[r1], so1)
    out0.wait()
    out1.wait()


@jax.jit
def kernel(x):
    b, n = x.shape
    mesh = plsc.VectorSubcoreMesh(
        core_axis_name="c", subcore_axis_name="s", num_cores=2,
        num_subcores=16)
    run = functools.partial(
        pl.kernel,
        out_type=jax.ShapeDtypeStruct((b, n), jnp.float32),
        mesh=mesh,
        compiler_params=pltpu.CompilerParams(needs_layout_passes=False),
        scratch_types=[
            pltpu.VMEM((n,), jnp.float32),
            pltpu.VMEM((n,), jnp.float32),
            pltpu.VMEM((2048,), jnp.int32),
            pltpu.SemaphoreType.DMA,
            pltpu.SemaphoreType.DMA,
            pltpu.SemaphoreType.DMA,
            pltpu.SemaphoreType.DMA,
        ],
    )(_sc_topk_body)
    return run(x)
